# baseline (speedup 1.0000x reference)
import functools

import jax
import jax.numpy as jnp
from jax.experimental import pallas as pl
from jax.experimental.pallas import tpu as pltpu

N = 2048
D = 256
K1 = 11
BLK = 512
NB = N // BLK
_BIG = 3.4e38


def _fused_kernel(nodes_ref, out_ref, s_s, thrc_s, thrr_s, degc_s, degr_s):
    p = pl.program_id(0)
    i = pl.program_id(1)
    row = pl.ds(i * BLK, BLK)

    @pl.when(p == 0)
    def _phase_s_thr():
        nodes = nodes_ref[...]
        blk = nodes_ref[row, :]
        sq = jnp.sum(nodes * nodes, axis=1)
        sq_blk = jnp.sum(blk * blk, axis=1)
        g = jax.lax.dot_general(
            blk, nodes, (((1,), (1,)), ((), ())),
            preferred_element_type=jnp.float32,
            precision=jax.lax.Precision.DEFAULT,
        )
        s = sq_blk[:, None] + sq[None, :] - 2.0 * g
        s = jnp.maximum(s, 0.0)
        gi = jax.lax.broadcasted_iota(jnp.int32, (BLK, N), 0) + i * BLK
        gj = jax.lax.broadcasted_iota(jnp.int32, (BLK, N), 1)
        s = jnp.where(gi == gj, 0.0, s)
        s_s[row, :] = s
        cur = s
        for _ in range(K1 - 1):
            m = jnp.min(cur, axis=1, keepdims=True)
            cur = jnp.where(cur == m, _BIG, cur)
        thr = jnp.min(cur, axis=1, keepdims=True)
        thrc_s[row, :] = thr
        ti = jax.lax.broadcasted_iota(jnp.int32, (BLK, BLK), 0)
        tj = jax.lax.broadcasted_iota(jnp.int32, (BLK, BLK), 1)
        diag = jnp.where(ti == tj, thr, _BIG)
        thrr_s[0:1, pl.ds(i * BLK, BLK)] = jnp.min(diag, axis=0, keepdims=True)

    @pl.when(p == 1)
    def _phase_a_deg():
        s = s_s[row, :]
        thr_i = thrc_s[row, :]
        thr_j = thrr_s[...]
        a = jnp.where(s <= jnp.maximum(thr_i, thr_j), jnp.exp(s * -0.5), 0.0)
        degc_s[row, :] = jnp.sum(a, axis=1, keepdims=True)

        @pl.when(i == 0)
        def _():
            degr_s[...] = jnp.sum(a, axis=0, keepdims=True)

        @pl.when(i != 0)
        def _():
            degr_s[...] += jnp.sum(a, axis=0, keepdims=True)

    @pl.when(p == 2)
    def _phase_lsym():
        s = s_s[row, :]
        thr_i = thrc_s[row, :]
        thr_j = thrr_s[...]
        a = jnp.where(s <= jnp.maximum(thr_i, thr_j), jnp.exp(s * -0.5), 0.0)
        degi = degc_s[row, :]
        dinv_i = 1.0 / jnp.sqrt(degi)
        dinv_j = 1.0 / jnp.sqrt(degr_s[...])
        gi = jax.lax.broadcasted_iota(jnp.int32, (BLK, N), 0) + i * BLK
        gj = jax.lax.broadcasted_iota(jnp.int32, (BLK, N), 1)
        l = jnp.where(gi == gj, degi - a, -a)
        m1 = (dinv_i * l) * dinv_j
        m2 = (dinv_j * l) * dinv_i
        out_ref[...] = 0.5 * (m1 + m2)


@functools.partial(jax.jit, static_argnames=("interpret",))
def _build_lsym(nodes, interpret=False):
    return pl.pallas_call(
        _fused_kernel,
        grid=(3, NB),
        in_specs=[pl.BlockSpec((N, D), lambda p, i: (0, 0))],
        out_specs=pl.BlockSpec((BLK, N), lambda p, i: (jnp.where(p == 2, i, 0), 0)),
        out_shape=jax.ShapeDtypeStruct((N, N), jnp.float32),
        scratch_shapes=[
            pltpu.VMEM((N, N), jnp.float32),
            pltpu.VMEM((N, 1), jnp.float32),
            pltpu.VMEM((1, N), jnp.float32),
            pltpu.VMEM((N, 1), jnp.float32),
            pltpu.VMEM((1, N), jnp.float32),
        ],
        interpret=interpret,
    )(nodes)


def kernel(nodes, labels):
    lsym = _build_lsym(nodes)
    _, evecs = jnp.linalg.eigh(lsym)
    return evecs


# speedup vs baseline: 1.0167x; 1.0167x over previous
import functools

import jax
import jax.numpy as jnp
from jax.experimental import pallas as pl
from jax.experimental.pallas import tpu as pltpu

N = 2048
D = 256
K1 = 11
BLK = 512
NB = N // BLK
_BIG = 3.4e38


def _fused_kernel(nodes_ref, out_ref, s_s, thrc_s, thrr_s, degc_s, degr_s):
    p = pl.program_id(0)
    i = pl.program_id(1)
    row = pl.ds(i * BLK, BLK)

    @pl.when(p == 0)
    def _phase_s_thr():
        nodes = nodes_ref[...]
        blk = nodes_ref[row, :]
        sq = jnp.sum(nodes * nodes, axis=1)
        sq_blk = jnp.sum(blk * blk, axis=1)
        g = jax.lax.dot_general(
            blk, nodes, (((1,), (1,)), ((), ())),
            preferred_element_type=jnp.float32,
            precision=jax.lax.Precision.DEFAULT,
        )
        s = sq_blk[:, None] + sq[None, :] - 2.0 * g
        s = jnp.maximum(s, 0.0)
        gi = jax.lax.broadcasted_iota(jnp.int32, (BLK, N), 0) + i * BLK
        gj = jax.lax.broadcasted_iota(jnp.int32, (BLK, N), 1)
        s = jnp.where(gi == gj, 0.0, s)
        s_s[row, :] = s
        cur = jnp.where(gi == gj, _BIG, s)
        m = jnp.min(cur, axis=1, keepdims=True)
        for _ in range(K1 - 2):
            m = jnp.min(jnp.where(cur > m, cur, _BIG), axis=1, keepdims=True)
        thr = m
        thrc_s[row, :] = thr
        ti = jax.lax.broadcasted_iota(jnp.int32, (BLK, BLK), 0)
        tj = jax.lax.broadcasted_iota(jnp.int32, (BLK, BLK), 1)
        diag = jnp.where(ti == tj, thr, _BIG)
        thrr_s[0:1, pl.ds(i * BLK, BLK)] = jnp.min(diag, axis=0, keepdims=True)

    @pl.when(p == 1)
    def _phase_a_deg():
        s = s_s[row, :]
        thr_i = thrc_s[row, :]
        thr_j = thrr_s[...]
        a = jnp.where(s <= jnp.maximum(thr_i, thr_j), jnp.exp(s * -0.5), 0.0)
        s_s[row, :] = a
        degc_s[row, :] = jnp.sum(a, axis=1, keepdims=True)

        @pl.when(i == 0)
        def _():
            degr_s[...] = jnp.sum(a, axis=0, keepdims=True)

        @pl.when(i != 0)
        def _():
            degr_s[...] += jnp.sum(a, axis=0, keepdims=True)

    @pl.when(p == 2)
    def _phase_lsym():
        a = s_s[row, :]
        degi = degc_s[row, :]
        dinv_i = 1.0 / jnp.sqrt(degi)
        dinv_j = 1.0 / jnp.sqrt(degr_s[...])
        gi = jax.lax.broadcasted_iota(jnp.int32, (BLK, N), 0) + i * BLK
        gj = jax.lax.broadcasted_iota(jnp.int32, (BLK, N), 1)
        l = jnp.where(gi == gj, degi - a, -a)
        m1 = (dinv_i * l) * dinv_j
        m2 = (dinv_j * l) * dinv_i
        out_ref[...] = 0.5 * (m1 + m2)


@functools.partial(jax.jit, static_argnames=("interpret",))
def _build_lsym(nodes, interpret=False):
    return pl.pallas_call(
        _fused_kernel,
        grid=(3, NB),
        in_specs=[pl.BlockSpec((N, D), lambda p, i: (0, 0))],
        out_specs=pl.BlockSpec((BLK, N), lambda p, i: (jnp.where(p == 2, i, 0), 0)),
        out_shape=jax.ShapeDtypeStruct((N, N), jnp.float32),
        scratch_shapes=[
            pltpu.VMEM((N, N), jnp.float32),
            pltpu.VMEM((N, 1), jnp.float32),
            pltpu.VMEM((1, N), jnp.float32),
            pltpu.VMEM((N, 1), jnp.float32),
            pltpu.VMEM((1, N), jnp.float32),
        ],
        interpret=interpret,
    )(nodes)


def kernel(nodes, labels):
    lsym = _build_lsym(nodes)
    _, evecs = jnp.linalg.eigh(lsym)
    return evecs


# speedup vs baseline: 1.0201x; 1.0033x over previous
import functools

import jax
import jax.numpy as jnp
from jax.experimental import pallas as pl
from jax.experimental.pallas import tpu as pltpu

N = 2048
D = 256
K1 = 11
BLK = 512
NB = N // BLK
_BIG = 3.4e38


def _fused_kernel(nodes_ref, out_ref, s_s, thrc_s, thrr_s, degc_s, degr_s):
    p = pl.program_id(0)
    i = pl.program_id(1)
    row = pl.ds(i * BLK, BLK)

    @pl.when(p == 0)
    def _phase_s_thr():
        nodes = nodes_ref[...]
        blk = nodes_ref[row, :]
        sq = jnp.sum(nodes * nodes, axis=1)
        sq_blk = jnp.sum(blk * blk, axis=1)
        g = jax.lax.dot_general(
            blk, nodes, (((1,), (1,)), ((), ())),
            preferred_element_type=jnp.float32,
            precision=jax.lax.Precision.DEFAULT,
        )
        s = sq_blk[:, None] + sq[None, :] - 2.0 * g
        s = jnp.maximum(s, 0.0)
        gi = jax.lax.broadcasted_iota(jnp.int32, (BLK, N), 0) + i * BLK
        gj = jax.lax.broadcasted_iota(jnp.int32, (BLK, N), 1)
        s = jnp.where(gi == gj, 0.0, s)
        s_s[row, :] = s
        cur = jnp.where(gi == gj, _BIG, s)
        m = jnp.min(cur, axis=1, keepdims=True)
        for _ in range(K1 - 2):
            m = jnp.min(jnp.where(cur > m, cur, _BIG), axis=1, keepdims=True)
        thr = m
        thrc_s[row, :] = thr
        ti = jax.lax.broadcasted_iota(jnp.int32, (BLK, BLK), 0)
        tj = jax.lax.broadcasted_iota(jnp.int32, (BLK, BLK), 1)
        diag = jnp.where(ti == tj, thr, _BIG)
        thrr_s[0:1, pl.ds(i * BLK, BLK)] = jnp.min(diag, axis=0, keepdims=True)

    @pl.when(p == 1)
    def _phase_a_deg():
        s = s_s[row, :]
        thr_i = thrc_s[row, :]
        thr_j = thrr_s[...]
        a = jnp.where(s <= jnp.maximum(thr_i, thr_j), jnp.exp(s * -0.5), 0.0)
        s_s[row, :] = a
        degc_s[row, :] = jnp.sum(a, axis=1, keepdims=True)

        @pl.when(i == 0)
        def _():
            degr_s[...] = jnp.sum(a, axis=0, keepdims=True)

        @pl.when(i != 0)
        def _():
            degr_s[...] += jnp.sum(a, axis=0, keepdims=True)

    @pl.when(p == 2)
    def _phase_lsym():
        a = s_s[row, :]
        degi = degc_s[row, :]
        dinv_i = 1.0 / jnp.sqrt(degi)
        dinv_j = 1.0 / jnp.sqrt(degr_s[...])
        gi = jax.lax.broadcasted_iota(jnp.int32, (BLK, N), 0) + i * BLK
        gj = jax.lax.broadcasted_iota(jnp.int32, (BLK, N), 1)
        l = jnp.where(gi == gj, degi - a, -a)
        out_ref[...] = (dinv_i * l) * dinv_j


@functools.partial(jax.jit, static_argnames=("interpret",))
def _build_lsym(nodes, interpret=False):
    return pl.pallas_call(
        _fused_kernel,
        grid=(3, NB),
        in_specs=[pl.BlockSpec((N, D), lambda p, i: (0, 0))],
        out_specs=pl.BlockSpec((BLK, N), lambda p, i: (jnp.where(p == 2, i, 0), 0)),
        out_shape=jax.ShapeDtypeStruct((N, N), jnp.float32),
        scratch_shapes=[
            pltpu.VMEM((N, N), jnp.float32),
            pltpu.VMEM((N, 1), jnp.float32),
            pltpu.VMEM((1, N), jnp.float32),
            pltpu.VMEM((N, 1), jnp.float32),
            pltpu.VMEM((1, N), jnp.float32),
        ],
        interpret=interpret,
    )(nodes)


def kernel(nodes, labels):
    lsym = _build_lsym(nodes)
    _, evecs = jnp.linalg.eigh(lsym)
    return evecs


# speedup vs baseline: 1.2101x; 1.1863x over previous
import functools

import jax
import jax.numpy as jnp
from jax.experimental import pallas as pl
from jax.experimental.pallas import tpu as pltpu

N = 2048
D = 256
K1 = 11
BLK = 512
NB = N // BLK
_BIG = 3.4e38


def _fused_kernel(nodes_ref, out_ref, s_s, thrc_s, thrr_s, degc_s, degr_s):
    p = pl.program_id(0)
    i = pl.program_id(1)
    row = pl.ds(i * BLK, BLK)

    @pl.when(p == 0)
    def _phase_s_thr():
        nodes = nodes_ref[...]
        blk = nodes_ref[row, :]
        sq = jnp.sum(nodes * nodes, axis=1)
        sq_blk = jnp.sum(blk * blk, axis=1)
        g = jax.lax.dot_general(
            blk, nodes, (((1,), (1,)), ((), ())),
            preferred_element_type=jnp.float32,
            precision=jax.lax.Precision.DEFAULT,
        )
        s = sq_blk[:, None] + sq[None, :] - 2.0 * g
        s = jnp.maximum(s, 0.0)
        gi = jax.lax.broadcasted_iota(jnp.int32, (BLK, N), 0) + i * BLK
        gj = jax.lax.broadcasted_iota(jnp.int32, (BLK, N), 1)
        s = jnp.where(gi == gj, 0.0, s)
        s_s[row, :] = s
        cur = jnp.where(gi == gj, _BIG, s)
        m = jnp.min(cur, axis=1, keepdims=True)
        for _ in range(K1 - 2):
            m = jnp.min(jnp.where(cur > m, cur, _BIG), axis=1, keepdims=True)
        thr = m
        thrc_s[row, :] = thr
        ti = jax.lax.broadcasted_iota(jnp.int32, (BLK, BLK), 0)
        tj = jax.lax.broadcasted_iota(jnp.int32, (BLK, BLK), 1)
        diag = jnp.where(ti == tj, thr, _BIG)
        thrr_s[0:1, pl.ds(i * BLK, BLK)] = jnp.min(diag, axis=0, keepdims=True)

    @pl.when(p == 1)
    def _phase_a_deg():
        s = s_s[row, :]
        thr_i = thrc_s[row, :]
        thr_j = thrr_s[...]
        a = jnp.where(s <= jnp.maximum(thr_i, thr_j), jnp.exp(s * -0.5), 0.0)
        s_s[row, :] = a
        degc_s[row, :] = jnp.sum(a, axis=1, keepdims=True)

        @pl.when(i == 0)
        def _():
            degr_s[...] = jnp.sum(a, axis=0, keepdims=True)

        @pl.when(i != 0)
        def _():
            degr_s[...] += jnp.sum(a, axis=0, keepdims=True)

    @pl.when(p == 2)
    def _phase_lsym():
        a = s_s[row, :]
        degi = degc_s[row, :]
        dinv_i = 1.0 / jnp.sqrt(degi)
        dinv_j = 1.0 / jnp.sqrt(degr_s[...])
        gi = jax.lax.broadcasted_iota(jnp.int32, (BLK, N), 0) + i * BLK
        gj = jax.lax.broadcasted_iota(jnp.int32, (BLK, N), 1)
        l = jnp.where(gi == gj, degi - a, -a)
        m1 = (dinv_i * l) * dinv_j
        m2 = (dinv_j * l) * dinv_i
        out_ref[...] = 0.5 * (m1 + m2)


@functools.partial(jax.jit, static_argnames=("interpret",))
def _build_lsym(nodes, interpret=False):
    return pl.pallas_call(
        _fused_kernel,
        grid=(3, NB),
        in_specs=[pl.BlockSpec((N, D), lambda p, i: (0, 0))],
        out_specs=pl.BlockSpec((BLK, N), lambda p, i: (jnp.where(p == 2, i, 0), 0)),
        out_shape=jax.ShapeDtypeStruct((N, N), jnp.float32),
        scratch_shapes=[
            pltpu.VMEM((N, N), jnp.float32),
            pltpu.VMEM((N, 1), jnp.float32),
            pltpu.VMEM((1, N), jnp.float32),
            pltpu.VMEM((N, 1), jnp.float32),
            pltpu.VMEM((1, N), jnp.float32),
        ],
        interpret=interpret,
    )(nodes)


def kernel(nodes, labels):
    lsym = _build_lsym(nodes)
    evecs, _ = jax.lax.linalg.eigh(lsym, symmetrize_input=False)
    return evecs
